# baseline (speedup 1.0000x reference)
import jax
import jax.numpy as jnp
from jax import lax
from jax.experimental import pallas as pl
from jax.experimental.pallas import tpu as pltpu
from jax.experimental.pallas import tpu_sc as plsc

V = 100000
H = 128
B = 64
T = 256
NTOK = B * T
TOT = 2 * NTOK
NW = 32

ROWS_PER_W = (2 * T) // NW


GDEPTH = 4


def _gather_body(tokT_hbm, emb_hbm, out_hbm, tok_v, rows_v, *sems):
    wid = lax.axis_index("c") * 16 + lax.axis_index("s")
    base = wid * ROWS_PER_W
    pltpu.sync_copy(tokT_hbm.at[pl.ds(base, ROWS_PER_W)], tok_v)
    copies = [None] * GDEPTH
    for i in range(GDEPTH - 1):
        copies[i] = pltpu.async_copy(
            emb_hbm.at[tok_v.at[i]], rows_v.at[i], sems[i])
    for i in range(ROWS_PER_W):
        j = i + GDEPTH - 1
        if j < ROWS_PER_W:
            copies[j % GDEPTH] = pltpu.async_copy(
                emb_hbm.at[tok_v.at[j]], rows_v.at[j % GDEPTH],
                sems[j % GDEPTH])
        copies[i % GDEPTH].wait()
        pltpu.sync_copy(rows_v.at[i % GDEPTH],
                        out_hbm.at[pl.ds((base + i) * B, B)])


def _make_gather():
    mesh = plsc.VectorSubcoreMesh(core_axis_name="c", subcore_axis_name="s")
    return pl.kernel(
        _gather_body,
        out_type=jax.ShapeDtypeStruct((TOT, H), jnp.float32),
        mesh=mesh,
        scratch_types=[
            pltpu.VMEM((ROWS_PER_W, B), jnp.int32),
            pltpu.VMEM((GDEPTH, B, H), jnp.float32),
        ] + [pltpu.SemaphoreType.DMA] * GDEPTH,
    )


VT = 4096
NT = pl.cdiv(V, VT)
NTFULL = V // VT
NBUF = 14
UNROLL = 8


def _mega_body(x_ref, y_ref,
               w0_ref, b0_ref, w1_ref, b1_ref,
               w2_ref, b2_ref, w3_ref, b3_ref,
               w_hbm, wtail_ref, bout_ref,
               out_ref, ys_ref, h_ref, wbuf, wsem):
    def sig(v):
        return 0.5 * jnp.tanh(0.5 * v) + 0.5

    def layer(src_ref, w_ref, b_ref, h0, store):
        cast = src_ref.dtype == jnp.float32

        def step(t, h):
            xt = src_ref[pl.ds(t * B, B), :]
            if cast:
                xt = xt.astype(jnp.bfloat16)
            xin = jnp.concatenate([xt, h.astype(jnp.bfloat16)], axis=1)
            g = jnp.dot(xin, w_ref[...],
                        preferred_element_type=jnp.float32) + b_ref[0, :]
            z = sig(g[:, :H])
            r = sig(g[:, H:2 * H])
            hh = jnp.tanh(g[:, 2 * H:3 * H] + r * g[:, 3 * H:])
            hn = hh + z * (h - hh)
            if store:
                ys_ref[pl.ds(t * B, B), :] = hn.astype(jnp.bfloat16)
            return hn

        def stepu(s, h):
            t = s * UNROLL
            for u in range(UNROLL):
                h = step(t + u, h)
            return h

        return lax.fori_loop(0, T // UNROLL, stepu, h0)

    i = pl.program_id(0)

    @pl.when(i == 0)
    def _():
        for s in range(NBUF):
            pltpu.make_async_copy(w_hbm.at[:, pl.ds(s * VT, VT)],
                                  wbuf.at[s], wsem.at[s]).start()
        h = jnp.zeros((B, H), jnp.float32)
        h = layer(x_ref, w0_ref, b0_ref, h, True)
        h = layer(ys_ref, w1_ref, b1_ref, h, False)
        h = layer(y_ref, w2_ref, b2_ref, h, True)
        h = layer(ys_ref, w3_ref, b3_ref, h, False)
        h_ref[...] = h

    @pl.when(jnp.logical_and(i >= 1, i + NBUF - 1 < NTFULL))
    def _():
        s = lax.rem(i - 1, NBUF)
        pltpu.make_async_copy(w_hbm.at[:, pl.ds((i + NBUF - 1) * VT, VT)],
                              wbuf.at[s], wsem.at[s]).start()

    @pl.when(i < NTFULL)
    def _():
        s = lax.rem(i, NBUF)
        pltpu.make_async_copy(w_hbm.at[:, pl.ds(i * VT, VT)],
                              wbuf.at[s], wsem.at[s]).wait()
        out_ref[...] = (
            jnp.dot(h_ref[...], wbuf[s], preferred_element_type=jnp.float32)
            + bout_ref[...])

    @pl.when(i == NTFULL)
    def _():
        out_ref[...] = (
            jnp.dot(h_ref[...], wtail_ref[...],
                    preferred_element_type=jnp.float32)
            + bout_ref[...])


def _run_mega(x_seq, y_seq, weights, Wout, bout):
    const = lambda i: (0, 0)
    full = pl.BlockSpec(index_map=const)
    wtail = jnp.pad(Wout[:, NTFULL * VT:], ((0, 0), (0, NT * VT - V)))
    in_specs = (
        [pl.BlockSpec((NTOK, H), const), pl.BlockSpec((NTOK, H), const)]
        + [full] * len(weights)
        + [pl.BlockSpec(memory_space=pl.ANY),
           pl.BlockSpec((H, VT), const),
           pl.BlockSpec((1, VT), lambda i: (0, i))]
    )
    return pl.pallas_call(
        _mega_body,
        grid=(NT,),
        in_specs=in_specs,
        out_specs=pl.BlockSpec((B, VT), lambda i: (0, i)),
        out_shape=jax.ShapeDtypeStruct((B, V), jnp.float32),
        scratch_shapes=[
            pltpu.VMEM((NTOK, H), jnp.bfloat16),
            pltpu.VMEM((B, H), jnp.float32),
            pltpu.VMEM((NBUF, H, VT), jnp.float32),
            pltpu.SemaphoreType.DMA((NBUF,)),
        ],
    )(x_seq, y_seq, *weights, Wout, wtail, bout.reshape(1, V))


def _combine(k, r, b):
    z = jnp.zeros((H, H), jnp.float32)
    top = jnp.concatenate([k[:, :2 * H], k[:, 2 * H:], z], axis=1)
    bot = jnp.concatenate([r[:, :2 * H], z, r[:, 2 * H:]], axis=1)
    w = jnp.concatenate([top, bot], axis=0).astype(jnp.bfloat16)
    bias = jnp.concatenate(
        [b[0, :2 * H] + b[1, :2 * H], b[0, 2 * H:], b[1, 2 * H:]]
    ).reshape(1, 4 * H)
    return w, bias


def kernel(encoder_tokens, decoder_tokens, emb,
           enc0_k, enc0_r, enc0_b, enc1_k, enc1_r, enc1_b,
           dec0_k, dec0_r, dec0_b, dec1_k, dec1_r, dec1_b,
           Wout, bout):
    tokT = jnp.concatenate(
        [encoder_tokens.T, decoder_tokens.T]
    ).astype(jnp.int32)
    rows = _make_gather()(tokT, emb)
    x_seq = rows[:NTOK]
    y_seq = rows[NTOK:]
    weights = (*_combine(enc0_k, enc0_r, enc0_b),
               *_combine(enc1_k, enc1_r, enc1_b),
               *_combine(dec0_k, dec0_r, dec0_b),
               *_combine(dec1_k, dec1_r, dec1_b))
    return _run_mega(x_seq, y_seq, weights, Wout, bout)


# speedup vs baseline: 1.0064x; 1.0064x over previous
import jax
import jax.numpy as jnp
from jax import lax
from jax.experimental import pallas as pl
from jax.experimental.pallas import tpu as pltpu
from jax.experimental.pallas import tpu_sc as plsc

V = 100000
H = 128
B = 64
T = 256
NTOK = B * T
TOT = 2 * NTOK
NW = 32

ROWS_PER_W = (2 * T) // NW


GDEPTH = 2


def _gather_body(tokT_hbm, emb_hbm, out_hbm, tok_v, rows_v, *sems):
    wid = lax.axis_index("c") * 16 + lax.axis_index("s")
    base = wid * ROWS_PER_W
    pltpu.sync_copy(tokT_hbm.at[pl.ds(base, ROWS_PER_W)], tok_v)
    copies = [None] * GDEPTH
    for i in range(GDEPTH - 1):
        copies[i] = pltpu.async_copy(
            emb_hbm.at[tok_v.at[i]], rows_v.at[i], sems[i])
    for i in range(ROWS_PER_W):
        j = i + GDEPTH - 1
        if j < ROWS_PER_W:
            copies[j % GDEPTH] = pltpu.async_copy(
                emb_hbm.at[tok_v.at[j]], rows_v.at[j % GDEPTH],
                sems[j % GDEPTH])
        copies[i % GDEPTH].wait()
        pltpu.sync_copy(rows_v.at[i % GDEPTH],
                        out_hbm.at[pl.ds((base + i) * B, B)])


def _make_gather():
    mesh = plsc.VectorSubcoreMesh(core_axis_name="c", subcore_axis_name="s")
    return pl.kernel(
        _gather_body,
        out_type=jax.ShapeDtypeStruct((TOT, H), jnp.float32),
        mesh=mesh,
        scratch_types=[
            pltpu.VMEM((ROWS_PER_W, B), jnp.int32),
            pltpu.VMEM((GDEPTH, B, H), jnp.float32),
        ] + [pltpu.SemaphoreType.DMA] * GDEPTH,
    )


VT = 4096
NT = pl.cdiv(V, VT)
NTFULL = V // VT
NBUF = 14
UNROLL = 8


def _mega_body(x_ref, y_ref,
               w0_ref, b0_ref, w1_ref, b1_ref,
               w2_ref, b2_ref, w3_ref, b3_ref,
               w_hbm, wtail_ref, bout_ref,
               out_ref, ys_ref, h_ref, wbuf, wsem):
    def sig(v):
        return 0.5 * jnp.tanh(0.5 * v) + 0.5

    def layer(src_ref, w_ref, b_ref, h0, store):
        cast = src_ref.dtype == jnp.float32

        def step(t, h):
            xt = src_ref[pl.ds(t * B, B), :]
            if cast:
                xt = xt.astype(jnp.bfloat16)
            xin = jnp.concatenate([xt, h.astype(jnp.bfloat16)], axis=1)
            g = jnp.dot(xin, w_ref[...],
                        preferred_element_type=jnp.float32) + b_ref[0, :]
            z = sig(g[:, :H])
            r = sig(g[:, H:2 * H])
            hh = jnp.tanh(g[:, 2 * H:3 * H] + r * g[:, 3 * H:])
            hn = hh + z * (h - hh)
            if store:
                ys_ref[pl.ds(t * B, B), :] = hn.astype(jnp.bfloat16)
            return hn

        def stepu(s, h):
            t = s * UNROLL
            for u in range(UNROLL):
                h = step(t + u, h)
            return h

        return lax.fori_loop(0, T // UNROLL, stepu, h0)

    i = pl.program_id(0)

    @pl.when(i == 0)
    def _():
        for s in range(NBUF):
            pltpu.make_async_copy(w_hbm.at[:, pl.ds(s * VT, VT)],
                                  wbuf.at[s], wsem.at[s]).start()
        h = jnp.zeros((B, H), jnp.float32)
        h = layer(x_ref, w0_ref, b0_ref, h, True)
        h = layer(ys_ref, w1_ref, b1_ref, h, False)
        h = layer(y_ref, w2_ref, b2_ref, h, True)
        h = layer(ys_ref, w3_ref, b3_ref, h, False)
        h_ref[...] = h

    @pl.when(jnp.logical_and(i >= 1, i + NBUF - 1 < NTFULL))
    def _():
        s = lax.rem(i - 1, NBUF)
        pltpu.make_async_copy(w_hbm.at[:, pl.ds((i + NBUF - 1) * VT, VT)],
                              wbuf.at[s], wsem.at[s]).start()

    @pl.when(i < NTFULL)
    def _():
        s = lax.rem(i, NBUF)
        pltpu.make_async_copy(w_hbm.at[:, pl.ds(i * VT, VT)],
                              wbuf.at[s], wsem.at[s]).wait()
        out_ref[...] = (
            jnp.dot(h_ref[...], wbuf[s], preferred_element_type=jnp.float32)
            + bout_ref[...])

    @pl.when(i == NTFULL)
    def _():
        out_ref[...] = (
            jnp.dot(h_ref[...], wtail_ref[...],
                    preferred_element_type=jnp.float32)
            + bout_ref[...])


def _run_mega(x_seq, y_seq, weights, Wout, bout):
    const = lambda i: (0, 0)
    full = pl.BlockSpec(index_map=const)
    wtail = jnp.pad(Wout[:, NTFULL * VT:], ((0, 0), (0, NT * VT - V)))
    in_specs = (
        [pl.BlockSpec((NTOK, H), const), pl.BlockSpec((NTOK, H), const)]
        + [full] * len(weights)
        + [pl.BlockSpec(memory_space=pl.ANY),
           pl.BlockSpec((H, VT), const),
           pl.BlockSpec((1, VT), lambda i: (0, i))]
    )
    return pl.pallas_call(
        _mega_body,
        grid=(NT,),
        in_specs=in_specs,
        out_specs=pl.BlockSpec((B, VT), lambda i: (0, i)),
        out_shape=jax.ShapeDtypeStruct((B, V), jnp.float32),
        scratch_shapes=[
            pltpu.VMEM((NTOK, H), jnp.bfloat16),
            pltpu.VMEM((B, H), jnp.float32),
            pltpu.VMEM((NBUF, H, VT), jnp.float32),
            pltpu.SemaphoreType.DMA((NBUF,)),
        ],
    )(x_seq, y_seq, *weights, Wout, wtail, bout.reshape(1, V))


def _combine(k, r, b):
    z = jnp.zeros((H, H), jnp.float32)
    top = jnp.concatenate([k[:, :2 * H], k[:, 2 * H:], z], axis=1)
    bot = jnp.concatenate([r[:, :2 * H], z, r[:, 2 * H:]], axis=1)
    w = jnp.concatenate([top, bot], axis=0).astype(jnp.bfloat16)
    bias = jnp.concatenate(
        [b[0, :2 * H] + b[1, :2 * H], b[0, 2 * H:], b[1, 2 * H:]]
    ).reshape(1, 4 * H)
    return w, bias


def kernel(encoder_tokens, decoder_tokens, emb,
           enc0_k, enc0_r, enc0_b, enc1_k, enc1_r, enc1_b,
           dec0_k, dec0_r, dec0_b, dec1_k, dec1_r, dec1_b,
           Wout, bout):
    tokT = jnp.concatenate(
        [encoder_tokens.T, decoder_tokens.T]
    ).astype(jnp.int32)
    rows = _make_gather()(tokT, emb)
    x_seq = rows[:NTOK]
    y_seq = rows[NTOK:]
    weights = (*_combine(enc0_k, enc0_r, enc0_b),
               *_combine(enc1_k, enc1_r, enc1_b),
               *_combine(dec0_k, dec0_r, dec0_b),
               *_combine(dec1_k, dec1_r, dec1_b))
    return _run_mega(x_seq, y_seq, weights, Wout, bout)


# speedup vs baseline: 1.0164x; 1.0099x over previous
import jax
import jax.numpy as jnp
from jax import lax
from jax.experimental import pallas as pl
from jax.experimental.pallas import tpu as pltpu
from jax.experimental.pallas import tpu_sc as plsc

V = 100000
H = 128
B = 64
T = 256
NTOK = B * T
TOT = 2 * NTOK
NW = 32

ROWS_PER_W = (2 * T) // NW


GDEPTH = 2


def _gather_body(tokT_hbm, emb_hbm, out_hbm, tok_v, rows_v, *sems):
    wid = lax.axis_index("c") * 16 + lax.axis_index("s")
    base = wid * ROWS_PER_W
    pltpu.sync_copy(tokT_hbm.at[pl.ds(base, ROWS_PER_W)], tok_v)
    copies = [None] * GDEPTH
    for i in range(GDEPTH - 1):
        copies[i] = pltpu.async_copy(
            emb_hbm.at[tok_v.at[i]], rows_v.at[i], sems[i])
    for i in range(ROWS_PER_W):
        j = i + GDEPTH - 1
        if j < ROWS_PER_W:
            copies[j % GDEPTH] = pltpu.async_copy(
                emb_hbm.at[tok_v.at[j]], rows_v.at[j % GDEPTH],
                sems[j % GDEPTH])
        copies[i % GDEPTH].wait()
        pltpu.sync_copy(rows_v.at[i % GDEPTH],
                        out_hbm.at[pl.ds((base + i) * B, B)])


def _make_gather():
    mesh = plsc.VectorSubcoreMesh(core_axis_name="c", subcore_axis_name="s")
    return pl.kernel(
        _gather_body,
        out_type=jax.ShapeDtypeStruct((TOT, H), jnp.float32),
        mesh=mesh,
        scratch_types=[
            pltpu.VMEM((ROWS_PER_W, B), jnp.int32),
            pltpu.VMEM((GDEPTH, B, H), jnp.float32),
        ] + [pltpu.SemaphoreType.DMA] * GDEPTH,
    )


VT = 4096
NT = pl.cdiv(V, VT)
NTFULL = V // VT
NBUF = 14
UNROLL = 8


def _mega_body(x_ref, y_ref,
               k0_ref, r0_ref, b0_ref, k1_ref, r1_ref, b1_ref,
               k2_ref, r2_ref, b2_ref, k3_ref, r3_ref, b3_ref,
               w_hbm, wtail_ref, bout_ref,
               out_ref, ys_ref, h_ref, wbuf, wsem, wg_ref, bg_ref):
    def sig(v):
        return 0.5 * jnp.tanh(0.5 * v) + 0.5

    def layer(src_ref, l, h0, store):
        cast = src_ref.dtype == jnp.float32

        def step(t, h):
            xt = src_ref[pl.ds(t * B, B), :]
            if cast:
                xt = xt.astype(jnp.bfloat16)
            xin = jnp.concatenate([xt, h.astype(jnp.bfloat16)], axis=1)
            g = jnp.dot(xin, wg_ref[l],
                        preferred_element_type=jnp.float32) + bg_ref[l]
            z = sig(g[:, :H])
            r = sig(g[:, H:2 * H])
            hh = jnp.tanh(g[:, 2 * H:3 * H] + r * g[:, 3 * H:])
            hn = hh + z * (h - hh)
            if store:
                ys_ref[pl.ds(t * B, B), :] = hn.astype(jnp.bfloat16)
            return hn

        def stepu(s, h):
            t = s * UNROLL
            for u in range(UNROLL):
                h = step(t + u, h)
            return h

        return lax.fori_loop(0, T // UNROLL, stepu, h0)

    i = pl.program_id(0)

    @pl.when(i == 0)
    def _():
        for s in range(NBUF):
            pltpu.make_async_copy(w_hbm.at[:, pl.ds(s * VT, VT)],
                                  wbuf.at[s], wsem.at[s]).start()
        bf = jnp.bfloat16
        zblk = jnp.zeros((H, H), bf)
        for l, (k_ref, r_ref, b_ref) in enumerate(
                [(k0_ref, r0_ref, b0_ref), (k1_ref, r1_ref, b1_ref),
                 (k2_ref, r2_ref, b2_ref), (k3_ref, r3_ref, b3_ref)]):
            wg_ref[l, :H, :3 * H] = k_ref[...].astype(bf)
            wg_ref[l, :H, 3 * H:] = zblk
            wg_ref[l, H:, :2 * H] = r_ref[:, :2 * H].astype(bf)
            wg_ref[l, H:, 2 * H:3 * H] = zblk
            wg_ref[l, H:, 3 * H:] = r_ref[:, 2 * H:].astype(bf)
            bg_ref[l, :2 * H] = b_ref[0, :2 * H] + b_ref[1, :2 * H]
            bg_ref[l, 2 * H:3 * H] = b_ref[0, 2 * H:]
            bg_ref[l, 3 * H:] = b_ref[1, 2 * H:]
        h = jnp.zeros((B, H), jnp.float32)
        h = layer(x_ref, 0, h, True)
        h = layer(ys_ref, 1, h, False)
        h = layer(y_ref, 2, h, True)
        h = layer(ys_ref, 3, h, False)
        h_ref[...] = h

    @pl.when(jnp.logical_and(i >= 1, i + NBUF - 1 < NTFULL))
    def _():
        s = lax.rem(i - 1, NBUF)
        pltpu.make_async_copy(w_hbm.at[:, pl.ds((i + NBUF - 1) * VT, VT)],
                              wbuf.at[s], wsem.at[s]).start()

    @pl.when(i < NTFULL)
    def _():
        s = lax.rem(i, NBUF)
        pltpu.make_async_copy(w_hbm.at[:, pl.ds(i * VT, VT)],
                              wbuf.at[s], wsem.at[s]).wait()
        out_ref[...] = (
            jnp.dot(h_ref[...], wbuf[s], preferred_element_type=jnp.float32)
            + bout_ref[...])

    @pl.when(i == NTFULL)
    def _():
        out_ref[...] = (
            jnp.dot(h_ref[...], wtail_ref[...],
                    preferred_element_type=jnp.float32)
            + bout_ref[...])


def _run_mega(x_seq, y_seq, weights, Wout, bout):
    const = lambda i: (0, 0)
    full = pl.BlockSpec(index_map=const)
    wtail = jnp.pad(Wout[:, NTFULL * VT:], ((0, 0), (0, NT * VT - V)))
    in_specs = (
        [pl.BlockSpec((NTOK, H), const), pl.BlockSpec((NTOK, H), const)]
        + [full] * len(weights)
        + [pl.BlockSpec(memory_space=pl.ANY),
           pl.BlockSpec((H, VT), const),
           pl.BlockSpec((1, VT), lambda i: (0, i))]
    )
    return pl.pallas_call(
        _mega_body,
        grid=(NT,),
        in_specs=in_specs,
        out_specs=pl.BlockSpec((B, VT), lambda i: (0, i)),
        out_shape=jax.ShapeDtypeStruct((B, V), jnp.float32),
        scratch_shapes=[
            pltpu.VMEM((NTOK, H), jnp.bfloat16),
            pltpu.VMEM((B, H), jnp.float32),
            pltpu.VMEM((NBUF, H, VT), jnp.float32),
            pltpu.SemaphoreType.DMA((NBUF,)),
            pltpu.VMEM((4, 2 * H, 4 * H), jnp.bfloat16),
            pltpu.VMEM((4, 4 * H), jnp.float32),
        ],
    )(x_seq, y_seq, *weights, Wout, wtail, bout.reshape(1, V))


def kernel(encoder_tokens, decoder_tokens, emb,
           enc0_k, enc0_r, enc0_b, enc1_k, enc1_r, enc1_b,
           dec0_k, dec0_r, dec0_b, dec1_k, dec1_r, dec1_b,
           Wout, bout):
    tokT = jnp.concatenate(
        [encoder_tokens.T, decoder_tokens.T]
    ).astype(jnp.int32)
    rows = _make_gather()(tokT, emb)
    x_seq = rows[:NTOK]
    y_seq = rows[NTOK:]
    weights = (enc0_k, enc0_r, enc0_b, enc1_k, enc1_r, enc1_b,
               dec0_k, dec0_r, dec0_b, dec1_k, dec1_r, dec1_b)
    return _run_mega(x_seq, y_seq, weights, Wout, bout)
